# baseline (speedup 1.0000x reference)
import functools

import jax
import jax.numpy as jnp
from jax.experimental import pallas as pl
from jax.experimental.pallas import tpu as pltpu

_HUFFMAN_PATHS = ((0, 1), (1, 0), (0, 0, 1), (1, 1, 0))
_NCOL = 16


def _coeff_table(n):
    r = jax.lax.broadcasted_iota(jnp.int32, (8, 2 * _NCOL), 0)
    c = jax.lax.broadcasted_iota(jnp.int32, (8, 2 * _NCOL), 1)
    t = jnp.zeros((8, 2 * _NCOL), jnp.float32)
    off = 0
    for w, path in enumerate(_HUFFMAN_PATHS):
        lw = len(path)
        coeff = 1.0 / (lw * n)
        t = jnp.where((r == w) & (c >= off) & (c < off + lw), coeff, t)
        ones = [off + j for j, bit in enumerate(path) if bit == 1]
        t = jnp.where(
            (r == w) & (c >= _NCOL + ones[0]) & (c < _NCOL + ones[-1] + 1),
            -coeff,
            t,
        )
        off += lw
    return t


def _make_body(batch, hdim, ch):
    nch = batch // ch

    def body(h_hbm, tw_ref, w0_ref, w1_ref, w2_ref, w3_ref, out_ref,
             wt_ref, wstk_ref, buf_ref, sem0, sem1):
        sems = (sem0, sem1)

        def chunk_copy(k):
            return pltpu.make_async_copy(
                h_hbm.at[pl.ds(k * ch, ch), :],
                buf_ref.at[k % 2],
                sems[k % 2],
            )

        chunk_copy(0).start()

        wstk_ref[...] = jnp.zeros_like(wstk_ref)
        wstk_ref[0:2, :] = w0_ref[...]
        wstk_ref[2:4, :] = w1_ref[...]
        wstk_ref[4:7, :] = w2_ref[...]
        wstk_ref[7:10, :] = w3_ref[...]
        eye = (
            jax.lax.broadcasted_iota(jnp.int32, (_NCOL, _NCOL), 0)
            == jax.lax.broadcasted_iota(jnp.int32, (_NCOL, _NCOL), 1)
        ).astype(jnp.float32)
        wt = jax.lax.dot_general(
            wstk_ref[...], eye, (((0,), (0,)), ((), ())),
            preferred_element_type=jnp.float32,
        )

        acc = jnp.zeros((8, 2 * _NCOL), jnp.float32)
        for k in range(nch):
            if k + 1 < nch:
                chunk_copy(k + 1).start()
            chunk_copy(k).wait()
            h = buf_ref[k % 2]
            tw = tw_ref[pl.ds(k * ch, ch), :]
            x = jnp.dot(h, wt, preferred_element_type=jnp.float32)
            soft = jnp.maximum(x, 0.0) + jnp.log1p(jnp.exp(-jnp.abs(x)))
            combo = jnp.concatenate([soft, x], axis=1)
            onehot = (
                tw == jax.lax.broadcasted_iota(jnp.int32, (ch, 8), 1)
            ).astype(jnp.float32)
            acc = acc + jax.lax.dot_general(
                onehot, combo, (((0,), (0,)), ((), ())),
                preferred_element_type=jnp.float32,
            )
        out_ref[0, 0] = jnp.sum(_coeff_table(batch) * acc)

    return body


@functools.partial(jax.jit, static_argnames=("interpret", "ch"))
def kernel(hidden, target_words, W_0, W_1, W_2, W_3, interpret=False, ch=1024):
    batch, hdim = hidden.shape
    tw2d = target_words.astype(jnp.int32).reshape(batch, 1)

    full = lambda shape: pl.BlockSpec(shape, lambda: (0, 0))
    out = pl.pallas_call(
        _make_body(batch, hdim, ch),
        in_specs=[
            pl.BlockSpec(memory_space=pl.ANY),
            full(tw2d.shape),
            full(W_0.shape),
            full(W_1.shape),
            full(W_2.shape),
            full(W_3.shape),
        ],
        out_specs=pl.BlockSpec(
            (1, 1), lambda: (0, 0), memory_space=pltpu.SMEM
        ),
        out_shape=jax.ShapeDtypeStruct((1, 1), jnp.float32),
        scratch_shapes=[
            pltpu.VMEM((hdim, _NCOL), jnp.float32),
            pltpu.VMEM((_NCOL, hdim), jnp.float32),
            pltpu.VMEM((2, ch, hdim), jnp.float32),
            pltpu.SemaphoreType.DMA,
            pltpu.SemaphoreType.DMA,
        ],
        interpret=interpret,
    )(hidden, tw2d, W_0, W_1, W_2, W_3)
    return out[0, 0]


# speedup vs baseline: 1.1929x; 1.1929x over previous
import functools

import jax
import jax.numpy as jnp
from jax.experimental import pallas as pl
from jax.experimental.pallas import tpu as pltpu

_HUFFMAN_PATHS = ((0, 1), (1, 0), (0, 0, 1), (1, 1, 0))
_NCOL = 16


def _coeff_tables(n):
    r = jax.lax.broadcasted_iota(jnp.int32, (8, _NCOL), 0)
    c = jax.lax.broadcasted_iota(jnp.int32, (8, _NCOL), 1)
    a = jnp.zeros((8, _NCOL), jnp.float32)
    b = jnp.zeros((8, _NCOL), jnp.float32)
    off = 0
    for w, path in enumerate(_HUFFMAN_PATHS):
        lw = len(path)
        coeff = 1.0 / (lw * n)
        a = jnp.where((r == w) & (c >= off) & (c < off + lw), coeff, a)
        ones = [off + j for j, bit in enumerate(path) if bit == 1]
        b = jnp.where(
            (r == w) & (c >= ones[0]) & (c < ones[-1] + 1), coeff, b
        )
        off += lw
    return a, b


def _body(h_ref, tw_ref, w0_ref, w1_ref, w2_ref, w3_ref, out_ref, wt_ref, wstk_ref):
    bm = h_ref.shape[0]
    n = pl.num_programs(0) * bm

    @pl.when(pl.program_id(0) == 0)
    def _():
        wstk_ref[...] = jnp.zeros_like(wstk_ref)
        wstk_ref[0:2, :] = w0_ref[...]
        wstk_ref[2:4, :] = w1_ref[...]
        wstk_ref[4:7, :] = w2_ref[...]
        wstk_ref[7:10, :] = w3_ref[...]
        eye = (
            jax.lax.broadcasted_iota(jnp.int32, (_NCOL, _NCOL), 0)
            == jax.lax.broadcasted_iota(jnp.int32, (_NCOL, _NCOL), 1)
        ).astype(jnp.float32)
        wt_ref[...] = jax.lax.dot_general(
            wstk_ref[...], eye, (((0,), (0,)), ((), ())),
            preferred_element_type=jnp.float32,
        )
        out_ref[0, 0] = 0.0

    h = h_ref[...]
    tw = tw_ref[...]
    x = jnp.dot(h, wt_ref[...], preferred_element_type=jnp.float32)
    soft = jnp.maximum(x, 0.0) + jnp.log1p(jnp.exp(-jnp.abs(x)))
    onehot = (tw == jax.lax.broadcasted_iota(jnp.int32, (bm, 8), 1)).astype(
        jnp.float32
    )
    s_tab = jax.lax.dot_general(
        onehot, soft, (((0,), (0,)), ((), ())),
        preferred_element_type=jnp.float32,
    )
    x_tab = jax.lax.dot_general(
        onehot, x, (((0,), (0,)), ((), ())),
        preferred_element_type=jnp.float32,
    )
    a_tab, b_tab = _coeff_tables(n)
    out_ref[0, 0] += jnp.sum(a_tab * s_tab - b_tab * x_tab)


@functools.partial(jax.jit, static_argnames=("interpret", "bm"))
def kernel(hidden, target_words, W_0, W_1, W_2, W_3, interpret=False, bm=2048):
    batch, hdim = hidden.shape
    grid = batch // bm
    tw2d = target_words.astype(jnp.int32).reshape(batch, 1)

    full = lambda shape: pl.BlockSpec(shape, lambda i: (0, 0))
    out = pl.pallas_call(
        _body,
        grid=(grid,),
        in_specs=[
            pl.BlockSpec((bm, hdim), lambda i: (i, 0)),
            pl.BlockSpec((bm, 1), lambda i: (i, 0)),
            full(W_0.shape),
            full(W_1.shape),
            full(W_2.shape),
            full(W_3.shape),
        ],
        out_specs=pl.BlockSpec(
            (1, 1), lambda i: (0, 0), memory_space=pltpu.SMEM
        ),
        out_shape=jax.ShapeDtypeStruct((1, 1), jnp.float32),
        scratch_shapes=[
            pltpu.VMEM((hdim, _NCOL), jnp.float32),
            pltpu.VMEM((_NCOL, hdim), jnp.float32),
        ],
        interpret=interpret,
    )(hidden, tw2d, W_0, W_1, W_2, W_3)
    return out[0, 0]
